# baseline (device time: 71760 ns/iter reference)
import jax
import jax.numpy as jnp
from jax import lax
from jax.experimental import pallas as pl
from jax.experimental.pallas import tpu as pltpu

N_DEV = 4
PAD = 576
CR = 8


def kernel(x, dest):
    m, n = x.shape

    x_bf = x.astype(jnp.bfloat16)
    ohd = (dest.reshape(m, 1) == jnp.arange(N_DEV, dtype=dest.dtype)
           .reshape(1, N_DEV)).astype(jnp.int32)
    cex = jnp.cumsum(ohd, axis=0) - ohd
    pos = jnp.sum(ohd * cex, axis=1)
    tvec = (dest.astype(jnp.int32) * PAD + pos).reshape(1, m)
    cnt = jnp.sum(ohd, axis=0).astype(jnp.int32)
    mycnt = jnp.pad(cnt.reshape(1, N_DEV), ((0, CR - 1), (0, 128 - N_DEV)))

    def body(x_ref, tvec_ref, mycnt_ref, out_ref,
             sendbuf_ref, staging_ref, gcnt_ref,
             send_x, recv_x, send_c, recv_c):
        my_x = lax.axis_index("x")
        my_y = lax.axis_index("y")
        my_z = lax.axis_index("z")

        barrier_sem = pltpu.get_barrier_semaphore()
        for d in range(1, N_DEV):
            pl.semaphore_signal(
                barrier_sem, inc=1,
                device_id=(my_x, (my_y + d) % N_DEV, my_z),
                device_id_type=pl.DeviceIdType.MESH,
            )
        pl.semaphore_wait(barrier_sem, N_DEV - 1)

        gcnt_ref[pl.ds(my_y * CR, CR), :] = mycnt_ref[:, :]
        csends = []
        for d in range(1, N_DEV):
            tgt = (my_y + d) % N_DEV
            dc = pltpu.make_async_remote_copy(
                src_ref=mycnt_ref,
                dst_ref=gcnt_ref.at[pl.ds(my_y * CR, CR)],
                send_sem=send_c.at[d],
                recv_sem=recv_c.at[d],
                device_id=(my_x, tgt, my_z),
                device_id_type=pl.DeviceIdType.MESH,
            )
            dc.start()
            csends.append(dc)

        tv = tvec_ref[0:1, :]
        xsends = []
        for d in range(1, N_DEV):
            tgt = (my_y + d) % N_DEV
            rows = lax.broadcasted_iota(jnp.int32, (PAD, m), 0) + tgt * PAD
            oh = (tv == rows).astype(jnp.bfloat16)
            sendbuf_ref[pl.ds(tgt * PAD, PAD), :] = jnp.dot(
                oh, x_ref[:, :], preferred_element_type=jnp.float32
            ).astype(jnp.bfloat16)
            dx = pltpu.make_async_remote_copy(
                src_ref=sendbuf_ref.at[pl.ds(tgt * PAD, PAD)],
                dst_ref=staging_ref.at[pl.ds(my_y * PAD, PAD)],
                send_sem=send_x.at[d],
                recv_sem=recv_x.at[d],
                device_id=(my_x, tgt, my_z),
                device_id_type=pl.DeviceIdType.MESH,
            )
            dx.start()
            xsends.append(dx)
        rows = lax.broadcasted_iota(jnp.int32, (PAD, m), 0) + my_y * PAD
        oh = (tv == rows).astype(jnp.bfloat16)
        staging_ref[pl.ds(my_y * PAD, PAD), :] = jnp.dot(
            oh, x_ref[:, :], preferred_element_type=jnp.float32
        ).astype(jnp.bfloat16)

        for d in range(1, N_DEV):
            s = (my_y - d) % N_DEV
            pltpu.make_async_remote_copy(
                src_ref=mycnt_ref,
                dst_ref=gcnt_ref.at[pl.ds(s * CR, CR)],
                send_sem=send_c.at[d],
                recv_sem=recv_c.at[d],
                device_id=(my_x, s, my_z),
                device_id_type=pl.DeviceIdType.MESH,
            ).wait_recv()

        g = gcnt_ref[:, :]
        colsel = (
            lax.broadcasted_iota(jnp.int32, (N_DEV * CR, 128), 1) == my_y
        )
        rowvals = jnp.sum(g * colsel, axis=1)
        cvec = rowvals.reshape(N_DEV, CR)[:, 0]
        tri = (
            lax.broadcasted_iota(jnp.int32, (N_DEV, N_DEV), 0)
            <= lax.broadcasted_iota(jnp.int32, (N_DEV, N_DEV), 1)
        ).astype(jnp.int32)
        cincl = jnp.sum(cvec.reshape(N_DEV, 1) * tri, axis=0).reshape(1, N_DEV)
        cexcl = cincl - cvec.reshape(1, N_DEV)
        i_col = lax.broadcasted_iota(jnp.int32, (m, 1), 0)
        ge = (i_col >= cincl).astype(jnp.int32)
        s_of_i = jnp.sum(ge, axis=1, keepdims=True)
        svals = lax.broadcasted_iota(jnp.int32, (1, N_DEV), 1)
        cexcl_of_i = jnp.sum(
            (s_of_i == svals).astype(jnp.int32) * cexcl, axis=1, keepdims=True
        )
        flat = s_of_i * PAD + (i_col - cexcl_of_i)
        segcols = lax.broadcasted_iota(jnp.int32, (m, PAD), 1)

        def seg_term(s):
            oh = (flat == segcols + s * PAD).astype(jnp.bfloat16)
            return jnp.dot(
                oh,
                staging_ref[pl.ds(s * PAD, PAD), :],
                preferred_element_type=jnp.float32,
            )

        out_ref[:, :] = seg_term(my_y)
        for d in range(1, N_DEV):
            s = (my_y - d) % N_DEV
            pltpu.make_async_remote_copy(
                src_ref=sendbuf_ref.at[pl.ds(0, PAD)],
                dst_ref=staging_ref.at[pl.ds(s * PAD, PAD)],
                send_sem=send_x.at[d],
                recv_sem=recv_x.at[d],
                device_id=(my_x, s, my_z),
                device_id_type=pl.DeviceIdType.MESH,
            ).wait_recv()
            out_ref[:, :] = out_ref[:, :] + seg_term(s)

        for dc in csends:
            dc.wait_send()
        for dx in xsends:
            dx.wait_send()

    out = pl.pallas_call(
        body,
        out_shape=jax.ShapeDtypeStruct((m, n), jnp.float32),
        in_specs=[
            pl.BlockSpec(memory_space=pltpu.VMEM),
            pl.BlockSpec(memory_space=pltpu.VMEM),
            pl.BlockSpec(memory_space=pltpu.VMEM),
        ],
        out_specs=pl.BlockSpec(memory_space=pltpu.VMEM),
        scratch_shapes=[
            pltpu.VMEM((N_DEV * PAD, n), jnp.bfloat16),
            pltpu.VMEM((N_DEV * PAD, n), jnp.bfloat16),
            pltpu.VMEM((N_DEV * CR, 128), jnp.int32),
            pltpu.SemaphoreType.DMA((N_DEV,)),
            pltpu.SemaphoreType.DMA((N_DEV,)),
            pltpu.SemaphoreType.DMA((N_DEV,)),
            pltpu.SemaphoreType.DMA((N_DEV,)),
        ],
        compiler_params=pltpu.CompilerParams(collective_id=0),
    )(x_bf, tvec, mycnt)
    return out
